# baseline (speedup 1.0000x reference)
import functools

import jax
import jax.numpy as jnp
from jax import lax
from jax.experimental import pallas as pl
from jax.experimental.pallas import tpu as pltpu
from jax.experimental.pallas import tpu_sc as plsc

BATCH = 16384
IDS_PER_ROW = 10
D = 64
B = BATCH * IDS_PER_ROW
NW = 32
B_PER_W = B // NW
NB = 8
RCHUNK = NB * IDS_PER_ROW
NCHUNK = B_PER_W // RCHUNK


def _emb_body(tab, team_tab, gids, rits, tids, out, gid_v, rit_v, tid_v,
              tiles_v, out_v, team_v, sem, sem2):
    c = lax.axis_index("c")
    s = lax.axis_index("s")
    wid = s * 2 + c
    base = pl.multiple_of(wid * B_PER_W, 128)
    b_base = pl.multiple_of(wid * (BATCH // NW), 8)

    pltpu.sync_copy(team_tab, team_v)
    pltpu.sync_copy(gids.at[pl.ds(base, B_PER_W)], gid_v)
    pltpu.sync_copy(rits.at[pl.ds(base, B_PER_W)], rit_v)
    pltpu.sync_copy(tids.at[pl.ds(base, B_PER_W)], tid_v)

    t0 = [team_v[0, pl.ds(g * 16, 16)] for g in range(4)]
    t1 = [team_v[1, pl.ds(g * 16, 16)] for g in range(4)]

    def chunk_body(ck, carry):
        off = ck * RCHUNK

        def issue(i, icarry):
            for u in range(2):
                rr = i * 2 + u
                splat = plsc.load_gather(
                    gid_v, [jnp.full((16,), off, jnp.int32) + rr]
                )
                grp8 = pl.multiple_of(jnp.max(splat) * 8, 8)
                pltpu.async_copy(
                    tab.at[pl.ds(grp8, 8)],
                    tiles_v.at[pl.ds(pl.multiple_of(rr * 8, 8), 8)],
                    sem,
                )
            return icarry

        lax.fori_loop(0, RCHUNK // 2, issue, 0)
        pltpu.make_async_copy(
            tab.at[pl.ds(0, RCHUNK * 8)], tiles_v, sem
        ).wait()

        lane = lax.iota(jnp.int32, 16)
        for rr in range(RCHUNK):
            osplat = jnp.full((16,), off, jnp.int32) + rr
            rsplat = plsc.load_gather(rit_v, [osplat])
            tv = plsc.load_gather(tid_v, [osplat])
            m = tv > 0
            srow = rsplat + rr * 8
            for g in range(4):
                val = plsc.load_gather(tiles_v, [srow, lane + 16 * g])
                add = jnp.where(m, t1[g], t0[g])
                out_v[rr, pl.ds(g * 16, 16)] = val + add

        pltpu.sync_copy(
            out_v, out.at[pl.ds(pl.multiple_of(base + ck * RCHUNK, 8), RCHUNK)]
        )
        return carry

    lax.fori_loop(0, NCHUNK, chunk_body, 0)


@jax.jit
def _emb_call(tab, team_tab, gids, rits, tids):
    kern = pl.kernel(
        _emb_body,
        out_type=jax.ShapeDtypeStruct((B, D), jnp.float32),
        mesh=plsc.VectorSubcoreMesh(core_axis_name="c", subcore_axis_name="s"),
        scratch_types=[
            pltpu.VMEM((B_PER_W,), jnp.int32),
            pltpu.VMEM((B_PER_W,), jnp.int32),
            pltpu.VMEM((B_PER_W,), jnp.int32),
            pltpu.VMEM((RCHUNK * 8, D), jnp.float32),
            pltpu.VMEM((RCHUNK, D), jnp.float32),
            pltpu.VMEM((2, D), jnp.float32),
            pltpu.SemaphoreType.DMA,
            pltpu.SemaphoreType.DMA,
        ],
        compiler_params=pltpu.CompilerParams(needs_layout_passes=False),
    )
    return kern(tab, team_tab, gids, rits, tids)


def kernel(numeric_features, champ_ids, team_ids, role_ids, subclass_ids,
           scaling_ids, champion_table, team_table):
    cids = champ_ids.reshape(-1).astype(jnp.int32)
    tids = team_ids.reshape(-1).astype(jnp.int32)
    gids = cids >> 3
    rits = cids & 7
    flat = _emb_call(champion_table, team_table, gids, rits, tids)
    return flat.reshape(BATCH, IDS_PER_ROW, D)


# speedup vs baseline: 1.3463x; 1.3463x over previous
import functools

import jax
import jax.numpy as jnp
from jax import lax
from jax.experimental import pallas as pl
from jax.experimental.pallas import tpu as pltpu
from jax.experimental.pallas import tpu_sc as plsc

BATCH = 16384
IDS_PER_ROW = 10
D = 64
B = BATCH * IDS_PER_ROW
NW = 32
B_PER_W = B // NW
RCHUNK = 40
NBC = RCHUNK // IDS_PER_ROW
NCHUNK = B_PER_W // RCHUNK
TROWS = (RCHUNK + 2) * 8

_LOADS = ((0, range(0, 16)), (16, range(0, 16)), (24, range(8, 16)))


def _emb_body(tab, team_tab, cids, tids, out, cid_v, tid_v, tiles_a, tiles_b,
              out_a, out_b, team_v, sem_a, sem_b, osem_a, osem_b):
    c = lax.axis_index("c")
    s = lax.axis_index("s")
    wid = s * 2 + c
    base = pl.multiple_of(wid * B_PER_W, 128)
    b_base = wid * (BATCH // NW)

    pltpu.sync_copy(team_tab, team_v)
    pltpu.sync_copy(cids.at[pl.ds(base, B_PER_W)], cid_v)
    pltpu.sync_copy(tids.at[pl.ds(base, B_PER_W)], tid_v)

    t0 = [team_v[0, pl.ds(g * 16, 16)] for g in range(4)]
    t1 = [team_v[1, pl.ds(g * 16, 16)] for g in range(4)]

    def issue(k, buf, sem):
        off = k * RCHUNK
        for j, lanes in _LOADS:
            v16 = cid_v[pl.ds(off + j, 16)]
            for u in lanes:
                rr = j + u
                cid = v16[u]
                grp8 = pl.multiple_of(cid & (-8), 8)
                dst = (rr + 1) * 8 - (cid & 7)
                pltpu.async_copy(
                    tab.at[pl.ds(grp8, 8)], buf.at[pl.ds(dst, 8)], sem
                )

    def process(k, buf, out_v, sem, osem):
        off = k * RCHUNK
        pltpu.make_async_copy(
            tab.at[pl.ds(0, RCHUNK * 8)], buf.at[pl.ds(0, RCHUNK * 8)], sem
        ).wait()

        @pl.when(k >= 2)
        def _():
            pltpu.make_async_copy(
                tab.at[pl.ds(0, RCHUNK)], out_v, osem
            ).wait()

        for j, lanes in _LOADS:
            t16 = tid_v[pl.ds(off + j, 16)]
            for u in lanes:
                rr = j + u
                sel = t16[u] > 0
                src = (rr + 1) * 8
                for g in range(4):
                    add = jnp.where(sel, t1[g], t0[g])
                    out_v[rr, pl.ds(g * 16, 16)] = (
                        buf[src, pl.ds(g * 16, 16)] + add
                    )
        b0 = b_base + k * NBC
        for bi in range(NBC):
            pltpu.async_copy(
                out_v.at[pl.ds(bi * IDS_PER_ROW, IDS_PER_ROW)],
                out.at[b0 + bi], osem,
            )

    issue(0, tiles_a, sem_a)

    def pair(i, carry):
        k0 = i * 2
        issue(k0 + 1, tiles_b, sem_b)
        process(k0, tiles_a, out_a, sem_a, osem_a)

        @pl.when(k0 + 2 < NCHUNK)
        def _():
            issue(k0 + 2, tiles_a, sem_a)

        process(k0 + 1, tiles_b, out_b, sem_b, osem_b)
        return carry

    lax.fori_loop(0, NCHUNK // 2, pair, 0)

    pltpu.make_async_copy(tab.at[pl.ds(0, RCHUNK)], out_a, osem_a).wait()
    pltpu.make_async_copy(tab.at[pl.ds(0, RCHUNK)], out_b, osem_b).wait()


@jax.jit
def _emb_call(tab, team_tab, cids, tids):
    kern = pl.kernel(
        _emb_body,
        out_type=jax.ShapeDtypeStruct((BATCH, IDS_PER_ROW, D), jnp.float32),
        mesh=plsc.VectorSubcoreMesh(core_axis_name="c", subcore_axis_name="s"),
        scratch_types=[
            pltpu.VMEM((B_PER_W,), jnp.int32),
            pltpu.VMEM((B_PER_W,), jnp.int32),
            pltpu.VMEM((TROWS, D), jnp.float32),
            pltpu.VMEM((TROWS, D), jnp.float32),
            pltpu.VMEM((RCHUNK, D), jnp.float32),
            pltpu.VMEM((RCHUNK, D), jnp.float32),
            pltpu.VMEM((2, D), jnp.float32),
            pltpu.SemaphoreType.DMA,
            pltpu.SemaphoreType.DMA,
            pltpu.SemaphoreType.DMA,
            pltpu.SemaphoreType.DMA,
        ],
    )
    return kern(tab, team_tab, cids, tids)


def kernel(numeric_features, champ_ids, team_ids, role_ids, subclass_ids,
           scaling_ids, champion_table, team_table):
    cids = champ_ids.reshape(-1).astype(jnp.int32)
    tids = team_ids.reshape(-1).astype(jnp.int32)
    return _emb_call(champion_table, team_table, cids, tids)


# speedup vs baseline: 1.3467x; 1.0003x over previous
import functools

import jax
import jax.numpy as jnp
from jax import lax
from jax.experimental import pallas as pl
from jax.experimental.pallas import tpu as pltpu
from jax.experimental.pallas import tpu_sc as plsc

BATCH = 16384
IDS_PER_ROW = 10
D = 64
B = BATCH * IDS_PER_ROW
NW = 32
B_PER_W = B // NW
RCHUNK = 40
NBC = RCHUNK // IDS_PER_ROW
NCHUNK = B_PER_W // RCHUNK
TROWS = (RCHUNK + 2) * 8

_LOADS = ((0, range(0, 16)), (16, range(0, 16)), (24, range(8, 16)))


def _emb_body(tab, team_tab, cids, tids, out, cid_v, tid_v, tiles_a, tiles_b,
              out_a, out_b, team_v, sem_a, sem_b, osem_a, osem_b):
    c = lax.axis_index("c")
    s = lax.axis_index("s")
    wid = s * 2 + c
    base = pl.multiple_of(wid * B_PER_W, 128)
    b_base = wid * (BATCH // NW)

    pltpu.sync_copy(team_tab, team_v)
    pltpu.sync_copy(cids.at[pl.ds(base, B_PER_W)], cid_v)
    pltpu.sync_copy(tids.at[pl.ds(base, B_PER_W)], tid_v)

    t0 = [team_v[0, pl.ds(g * 16, 16)] for g in range(4)]
    t1 = [team_v[1, pl.ds(g * 16, 16)] for g in range(4)]

    def issue(k, buf, sem):
        off = k * RCHUNK
        for j, lanes in _LOADS:
            v16 = cid_v[pl.ds(off + j, 16)]
            for u in lanes:
                rr = j + u
                cid = v16[u]
                grp8 = pl.multiple_of(cid & (-8), 8)
                dst = (rr + 1) * 8 - (cid & 7)
                pltpu.async_copy(
                    tab.at[pl.ds(grp8, 8)], buf.at[pl.ds(dst, 8)], sem
                )

    def process(k, buf, out_v, sem, osem):
        off = k * RCHUNK
        pltpu.make_async_copy(
            tab.at[pl.ds(0, RCHUNK * 8)], buf.at[pl.ds(0, RCHUNK * 8)], sem
        ).wait()

        @pl.when(k >= 2)
        def _():
            pltpu.make_async_copy(
                tab.at[pl.ds(0, RCHUNK)], out_v, osem
            ).wait()

        for j, lanes in _LOADS:
            t16 = tid_v[pl.ds(off + j, 16)]
            for u in lanes:
                rr = j + u
                sel = t16[u] > 0
                src = (rr + 1) * 8
                for g in range(4):
                    add = jnp.where(sel, t1[g], t0[g])
                    out_v[rr, pl.ds(g * 16, 16)] = (
                        buf[src, pl.ds(g * 16, 16)] + add
                    )
        b0 = b_base + k * NBC
        for bi in range(NBC):
            pltpu.async_copy(
                out_v.at[pl.ds(bi * IDS_PER_ROW, IDS_PER_ROW)],
                out.at[b0 + bi], osem,
            )

    issue(0, tiles_a, sem_a)

    def pair(i, carry):
        k0 = i * 2
        issue(k0 + 1, tiles_b, sem_b)
        process(k0, tiles_a, out_a, sem_a, osem_a)

        @pl.when(k0 + 2 < NCHUNK)
        def _():
            issue(k0 + 2, tiles_a, sem_a)

        process(k0 + 1, tiles_b, out_b, sem_b, osem_b)
        return carry

    lax.fori_loop(0, NCHUNK // 2, pair, 0)

    pltpu.make_async_copy(tab.at[pl.ds(0, RCHUNK)], out_a, osem_a).wait()
    pltpu.make_async_copy(tab.at[pl.ds(0, RCHUNK)], out_b, osem_b).wait()


@jax.jit
def _emb_call(tab, team_tab, cids, tids):
    kern = pl.kernel(
        _emb_body,
        out_type=jax.ShapeDtypeStruct((BATCH, IDS_PER_ROW, D), jnp.float32),
        mesh=plsc.VectorSubcoreMesh(core_axis_name="c", subcore_axis_name="s"),
        scratch_types=[
            pltpu.VMEM((B_PER_W,), jnp.int32),
            pltpu.VMEM((B_PER_W,), jnp.int32),
            pltpu.VMEM((TROWS, D), jnp.float32),
            pltpu.VMEM((TROWS, D), jnp.float32),
            pltpu.VMEM((RCHUNK, D), jnp.float32),
            pltpu.VMEM((RCHUNK, D), jnp.float32),
            pltpu.VMEM((2, D), jnp.float32),
            pltpu.SemaphoreType.DMA,
            pltpu.SemaphoreType.DMA,
            pltpu.SemaphoreType.DMA,
            pltpu.SemaphoreType.DMA,
        ],
        compiler_params=pltpu.CompilerParams(use_tc_tiling_on_sc=True),
    )
    return kern(tab, team_tab, cids, tids)


def kernel(numeric_features, champ_ids, team_ids, role_ids, subclass_ids,
           scaling_ids, champion_table, team_table):
    cids = champ_ids.reshape(-1).astype(jnp.int32)
    tids = team_ids.reshape(-1).astype(jnp.int32)
    return _emb_call(champion_table, team_table, cids, tids)
